# baseline (speedup 1.0000x reference)
import functools
import jax
import jax.numpy as jnp
from jax import lax
from jax.experimental import pallas as pl
from jax.experimental.pallas import tpu as pltpu

_TOP_K = 200
_CONF = 0.1
_NMS_T = 0.45
_V0 = 0.1
_V1 = 0.2
_LANES = 128
_CS = 2
_CH = _CS * _LANES


def _trans_body(conf_ref, oct_ref, ocl_ref, cmax_s,
                *, rowblk, nblk, num_classes):
    j = pl.program_id(1)
    pm = None
    for t in range(rowblk):
        tr = jnp.transpose(
            conf_ref[0, t * _LANES:(t + 1) * _LANES, :], (1, 0))
        oct_ref[0, :, 0, t] = tr
        pm = tr if pm is None else jnp.maximum(pm, tr)

    @pl.when(j == 0)
    def _():
        cmax_s[...] = pm

    @pl.when(j > 0)
    def _():
        cmax_s[...] = jnp.maximum(cmax_s[...], pm)

    @pl.when(j == nblk - 1)
    def _():
        cm = jnp.max(cmax_s[...], axis=1, keepdims=True)
        iota_c = lax.broadcasted_iota(jnp.int32, (num_classes, 1), 0)
        has = (cm > _CONF) & (iota_c >= 1)
        cl = jnp.min(jnp.where(has, iota_c, num_classes))
        clf = jnp.where(cl < num_classes, cl, -1)
        ocl_ref[0] = jnp.full((1, _LANES), clf, jnp.float32)


def _prep_body(cls_ref, srow_ref, loc_ref, pri_ref,
               om_ref, ox1_ref, oy1_ref, ox2_ref, oy2_ref,
               *, nch, num_classes):
    neg = jnp.float32(-jnp.inf)
    del cls_ref
    scores = srow_ref[0, 0]

    lx = loc_ref[0, 0]
    ly = loc_ref[0, 1]
    lw = loc_ref[0, 2]
    lh = loc_ref[0, 3]
    pcx = pri_ref[0]
    pcy = pri_ref[1]
    pw = pri_ref[2]
    ph = pri_ref[3]
    bcx = pcx + lx * _V0 * pw
    bcy = pcy + ly * _V0 * ph
    bw = pw * jnp.exp(lw * _V1)
    bh = ph * jnp.exp(lh * _V1)
    x1 = bcx - bw / 2
    y1 = bcy - bh / 2
    om_ref[0] = jnp.where(scores > _CONF, scores, neg).reshape(
        nch, _CS, _LANES)
    ox1_ref[0] = x1.reshape(nch, _CS, _LANES)
    oy1_ref[0] = y1.reshape(nch, _CS, _LANES)
    ox2_ref[0] = (bw + x1).reshape(nch, _CS, _LANES)
    oy2_ref[0] = (bh + y1).reshape(nch, _CS, _LANES)


def _detect_body(msk_ref, x1_ref, y1_ref, x2_ref, y2_ref, cl_ref,
                 os_ref, ox1_ref, oy1_ref, ox2_ref, oy2_ref,
                 sup_s, *msk_s,
                 b, nch, num_classes, top_k, slots):
    neg = jnp.float32(-jnp.inf)

    for i in range(b):
        msk_s[i][...] = msk_ref[i]
    cm0 = jnp.max(jnp.max(msk_ref[...], axis=3), axis=2)
    lane1 = lax.broadcasted_iota(jnp.int32, (1, nch), 1)
    row_ch = lax.broadcasted_iota(jnp.int32, (b, nch), 0)
    lane_ch = lax.broadcasted_iota(jnp.int32, (b, nch), 1)
    lin = (lax.broadcasted_iota(jnp.int32, (_CS, _LANES), 0) * _LANES
           + lax.broadcasted_iota(jnp.int32, (_CS, _LANES), 1))
    slot = lax.broadcasted_iota(jnp.int32, (b, slots), 1)
    fz = jnp.zeros((b, slots), jnp.float32)
    row1 = lax.broadcasted_iota(jnp.int32, (b, 1), 0)

    def ext_body(k, carry):
        cm, cs, c1, c2, c3, c4 = carry
        m_vec = jnp.max(cm, axis=1, keepdims=True)
        cbs = []
        for i in range(b):
            cbs.append(jnp.max(jnp.where(cm[i:i + 1] == m_vec[i:i + 1],
                                         lane1, -1)))
        chunks = [msk_s[i][pl.ds(cbs[i], 1)][0] for i in range(b)]
        boxc = [jnp.concatenate(
            [x1_ref[i, pl.ds(cbs[i], 1)],
             y1_ref[i, pl.ds(cbs[i], 1)],
             x2_ref[i, pl.ds(cbs[i], 1)],
             y2_ref[i, pl.ds(cbs[i], 1)]], axis=1) for i in range(b)]
        ohs = []
        news = []
        for i in range(b):
            liv = jnp.max(jnp.where(chunks[i] == m_vec[i:i + 1], lin, -1),
                          keepdims=True)
            oh = lin == liv
            ohs.append(oh)
            news.append(jnp.where(oh, neg, chunks[i]))
        vx1 = fz[:, :1]
        vy1 = fz[:, :1]
        vx2 = fz[:, :1]
        vy2 = fz[:, :1]
        cm_new = cm
        for i in range(b):
            bsel = row1 == i
            ohf = jnp.where(ohs[i], 1.0, 0.0)[None]
            oh4 = jnp.concatenate([ohf] * 4, axis=1)
            bsum = jnp.sum(oh4 * boxc[i],
                           axis=2, keepdims=True)
            bx1 = bsum[:, 0, :]
            by1 = bsum[:, _CS, :]
            bx2 = bsum[:, 2 * _CS, :]
            by2 = bsum[:, 3 * _CS, :]
            for t in range(1, _CS):
                bx1 = bx1 + bsum[:, t, :]
                by1 = by1 + bsum[:, _CS + t, :]
                bx2 = bx2 + bsum[:, 2 * _CS + t, :]
                by2 = by2 + bsum[:, 3 * _CS + t, :]
            vx1 = jnp.where(bsel, bx1, vx1)
            vy1 = jnp.where(bsel, by1, vy1)
            vx2 = jnp.where(bsel, bx2, vx2)
            vy2 = jnp.where(bsel, by2, vy2)
            nmxv = jnp.max(news[i], keepdims=True)
            cm_new = jnp.where((row_ch == i) & (lane_ch == cbs[i]),
                               nmxv, cm_new)
        for i in range(b):
            msk_s[i][pl.ds(cbs[i], 1)] = news[i][None]
        koh = slot == k
        cs = jnp.where(koh, m_vec, cs)
        c1 = jnp.where(koh, vx1, c1)
        c2 = jnp.where(koh, vy1, c2)
        c3 = jnp.where(koh, vx2, c3)
        c4 = jnp.where(koh, vy2, c4)
        return cm_new, cs, c1, c2, c3, c4

    _, cs, c1, c2, c3, c4 = lax.fori_loop(
        0, top_k, ext_body,
        (cm0, jnp.full((b, slots), neg), fz, fz, fz, fz))

    carea = (c3 - c1) * (c4 - c2)
    alive0 = jnp.where(cs > _CONF, 1, 0)
    x1T = c1[:, :, None]
    y1T = c2[:, :, None]
    x2T = c3[:, :, None]
    y2T = c4[:, :, None]
    aT = carea[:, :, None]
    x1B = c1[:, None, :]
    y1B = c2[:, None, :]
    x2B = c3[:, None, :]
    y2B = c4[:, None, :]
    aB = carea[:, None, :]
    ww = jnp.maximum(jnp.minimum(x2T, x2B) - jnp.maximum(x1T, x1B), 0.0)
    hh = jnp.maximum(jnp.minimum(y2T, y2B) - jnp.maximum(y1T, y1B), 0.0)
    inter = ww * hh
    iou = inter / ((aB - inter) + aT)
    sup_s[...] = jnp.where(iou <= _NMS_T, 0, 1)

    supp = jnp.zeros((b, slots), jnp.int32)
    kept = jnp.zeros((b, slots), jnp.int32)
    for s in range(slots):
        keep_s = jnp.where(
            (alive0[:, s:s + 1] > 0) & (supp[:, s:s + 1] == 0), 1, 0)
        supp = supp | jnp.where(keep_s > 0, sup_s[:, s], 0)
        kept = jnp.where(slot == s, keep_s, kept)

    pos = kept
    sh = 1
    while sh < slots:
        pos = pos + jnp.concatenate(
            [jnp.zeros((b, sh), jnp.int32), pos[:, :slots - sh]], axis=1)
        sh *= 2
    pos = pos - kept
    iota_r = lax.broadcasted_iota(jnp.int32, (b, slots, slots), 2)
    perm = jnp.where((pos[:, :, None] == iota_r) & (kept[:, :, None] > 0),
                     1.0, 0.0)
    csz = jnp.where(kept > 0, cs, 0.0)
    rs = jnp.sum(perm * csz[:, :, None], axis=1)
    r1 = jnp.sum(perm * c1[:, :, None], axis=1)
    r2 = jnp.sum(perm * c2[:, :, None], axis=1)
    r3 = jnp.sum(perm * c3[:, :, None], axis=1)
    r4 = jnp.sum(perm * c4[:, :, None], axis=1)

    clf = cl_ref[:, 0:1]
    cls = clf.astype(jnp.int32).reshape(b, 1, 1)
    found = (clf >= 0).reshape(b, 1, 1)
    cmask = (lax.broadcasted_iota(jnp.int32, (b, num_classes, 1), 1) == cls
             ) & found
    os_ref[...] = jnp.where(cmask, rs.reshape(b, 1, slots), 0.0)
    ox1_ref[...] = jnp.where(cmask, r1.reshape(b, 1, slots), 0.0)
    oy1_ref[...] = jnp.where(cmask, r2.reshape(b, 1, slots), 0.0)
    ox2_ref[...] = jnp.where(cmask, r3.reshape(b, 1, slots), 0.0)
    oy2_ref[...] = jnp.where(cmask, r4.reshape(b, 1, slots), 0.0)


@jax.jit
def kernel(loc_data, conf_data, prior_data):
    b, n, _ = loc_data.shape
    num_classes = conf_data.shape[2]
    npad = -(-n // _CH) * _CH
    rows = npad // _LANES
    nch = npad // _CH
    su = -(-_TOP_K // _LANES)
    slots = su * _LANES

    loc_t = jnp.transpose(loc_data, (0, 2, 1))
    pri_t = jnp.transpose(prior_data, (1, 0))
    pad = npad - n
    loc_t = jnp.pad(loc_t, ((0, 0), (0, 0), (0, pad)))
    pri_t = jnp.pad(pri_t, ((0, 0), (0, pad)))
    loc_t = loc_t.reshape(b, 4, rows, _LANES)
    pri_t = pri_t.reshape(4, rows, _LANES)
    conf_p = jnp.pad(conf_data, ((0, 0), (0, pad), (0, 0)))

    nblk = 2 if rows % 2 == 0 else 1
    rowblk = rows // nblk
    trans = functools.partial(_trans_body, rowblk=rowblk, nblk=nblk,
                              num_classes=num_classes)
    cl_sh = jax.ShapeDtypeStruct((b, 1, _LANES), jnp.float32)
    conf_t, clo = pl.pallas_call(
        trans,
        grid=(b, nblk),
        in_specs=[
            pl.BlockSpec((1, rowblk * _LANES, num_classes),
                         lambda i, j: (i, j, 0)),
        ],
        out_specs=[
            pl.BlockSpec((1, num_classes, 1, rowblk, _LANES),
                         lambda i, j: (i, 0, j, 0, 0)),
            pl.BlockSpec((1, 1, _LANES), lambda i, j: (i, 0, 0)),
        ],
        out_shape=[
            jax.ShapeDtypeStruct((b, num_classes, nblk, rowblk, _LANES),
                                 jnp.float32),
            cl_sh,
        ],
        scratch_shapes=[pltpu.VMEM((num_classes, _LANES), jnp.float32)],
    )(conf_p)
    conf_t = conf_t.reshape(b, num_classes, rows, _LANES)
    clv = clo.reshape(b, _LANES)
    cls_idx = jnp.where(clv[:, 0] >= 0, clv[:, 0], 1.0).astype(jnp.int32)

    prep = functools.partial(_prep_body, nch=nch, num_classes=num_classes)
    plane_sh = jax.ShapeDtypeStruct((b, nch, _CS, _LANES), jnp.float32)
    planes = pl.pallas_call(
        prep,
        grid_spec=pltpu.PrefetchScalarGridSpec(
            num_scalar_prefetch=1,
            grid=(b,),
            in_specs=[
                pl.BlockSpec((1, 1, rows, _LANES),
                             lambda i, cls: (i, cls[i], 0, 0)),
                pl.BlockSpec((1, 4, rows, _LANES),
                             lambda i, cls: (i, 0, 0, 0)),
                pl.BlockSpec((4, rows, _LANES), lambda i, cls: (0, 0, 0)),
            ],
            out_specs=[pl.BlockSpec((1, nch, _CS, _LANES),
                                    lambda i, cls: (i, 0, 0, 0))] * 5,
        ),
        out_shape=[plane_sh] * 5,
    )(cls_idx, conf_t, loc_t, pri_t)

    msk, x1p, y1p, x2p, y2p = planes[:5]

    det = functools.partial(_detect_body, b=b, nch=nch,
                            num_classes=num_classes, top_k=_TOP_K,
                            slots=slots)
    out_sh = jax.ShapeDtypeStruct((b, num_classes, slots), jnp.float32)
    outs = pl.pallas_call(
        det,
        out_shape=[out_sh] * 5,
        scratch_shapes=[pltpu.VMEM((b, slots, slots), jnp.int32)]
        + [pltpu.VMEM((nch, _CS, _LANES), jnp.float32)] * b,
    )(msk, x1p, y1p, x2p, y2p, clv)

    stacked = jnp.stack(outs, axis=-1)
    return stacked[:, :, :_TOP_K, :]


# speedup vs baseline: 1.0441x; 1.0441x over previous
import functools
import jax
import jax.numpy as jnp
from jax import lax
from jax.experimental import pallas as pl
from jax.experimental.pallas import tpu as pltpu

_TOP_K = 200
_CONF = 0.1
_NMS_T = 0.45
_V0 = 0.1
_V1 = 0.2
_LANES = 128
_CS = 2
_CH = _CS * _LANES


def _cmax_body(conf_ref, ocl_ref, cmax_s, *, nblk, num_classes):
    j = pl.program_id(1)
    pm = jnp.max(conf_ref[0], axis=0, keepdims=True)

    @pl.when(j == 0)
    def _():
        cmax_s[...] = pm

    @pl.when(j > 0)
    def _():
        cmax_s[...] = jnp.maximum(cmax_s[...], pm)

    @pl.when(j == nblk - 1)
    def _():
        cm = cmax_s[...]
        iota_c = lax.broadcasted_iota(jnp.int32, (1, num_classes), 1)
        has = (cm > _CONF) & (iota_c >= 1)
        cl = jnp.min(jnp.where(has, iota_c, num_classes))
        clf = jnp.where(cl < num_classes, cl, -1)
        ocl_ref[0] = jnp.full((1, _LANES), clf, jnp.float32)


def _prep_body(scr_ref, loc_ref, pri_ref,
               om_ref, ox1_ref, oy1_ref, ox2_ref, oy2_ref,
               *, nch, num_classes):
    neg = jnp.float32(-jnp.inf)
    scores = scr_ref[0, 0]

    lx = loc_ref[0, 0]
    ly = loc_ref[0, 1]
    lw = loc_ref[0, 2]
    lh = loc_ref[0, 3]
    pcx = pri_ref[0]
    pcy = pri_ref[1]
    pw = pri_ref[2]
    ph = pri_ref[3]
    bcx = pcx + lx * _V0 * pw
    bcy = pcy + ly * _V0 * ph
    bw = pw * jnp.exp(lw * _V1)
    bh = ph * jnp.exp(lh * _V1)
    x1 = bcx - bw / 2
    y1 = bcy - bh / 2
    om_ref[0] = jnp.where(scores > _CONF, scores, neg).reshape(
        nch, _CS, _LANES)
    ox1_ref[0] = x1.reshape(nch, _CS, _LANES)
    oy1_ref[0] = y1.reshape(nch, _CS, _LANES)
    ox2_ref[0] = (bw + x1).reshape(nch, _CS, _LANES)
    oy2_ref[0] = (bh + y1).reshape(nch, _CS, _LANES)


def _detect_body(msk_ref, x1_ref, y1_ref, x2_ref, y2_ref, cl_ref,
                 os_ref, ox1_ref, oy1_ref, ox2_ref, oy2_ref,
                 sup_s, *msk_s,
                 b, nch, num_classes, top_k, slots):
    neg = jnp.float32(-jnp.inf)

    for i in range(b):
        msk_s[i][...] = msk_ref[i]
    cm0 = jnp.max(jnp.max(msk_ref[...], axis=3), axis=2)
    lane1 = lax.broadcasted_iota(jnp.int32, (1, nch), 1)
    row_ch = lax.broadcasted_iota(jnp.int32, (b, nch), 0)
    lane_ch = lax.broadcasted_iota(jnp.int32, (b, nch), 1)
    lin = (lax.broadcasted_iota(jnp.int32, (_CS, _LANES), 0) * _LANES
           + lax.broadcasted_iota(jnp.int32, (_CS, _LANES), 1))
    slot = lax.broadcasted_iota(jnp.int32, (b, slots), 1)
    fz = jnp.zeros((b, slots), jnp.float32)
    row1 = lax.broadcasted_iota(jnp.int32, (b, 1), 0)

    def ext_body(k, carry):
        cm, cs, c1, c2, c3, c4 = carry
        m_vec = jnp.max(cm, axis=1, keepdims=True)
        cbs = []
        for i in range(b):
            cbs.append(jnp.max(jnp.where(cm[i:i + 1] == m_vec[i:i + 1],
                                         lane1, -1)))
        chunks = [msk_s[i][pl.ds(cbs[i], 1)][0] for i in range(b)]
        boxc = [jnp.concatenate(
            [x1_ref[i, pl.ds(cbs[i], 1)],
             y1_ref[i, pl.ds(cbs[i], 1)],
             x2_ref[i, pl.ds(cbs[i], 1)],
             y2_ref[i, pl.ds(cbs[i], 1)]], axis=1) for i in range(b)]
        ohs = []
        news = []
        for i in range(b):
            liv = jnp.max(jnp.where(chunks[i] == m_vec[i:i + 1], lin, -1),
                          keepdims=True)
            oh = lin == liv
            ohs.append(oh)
            news.append(jnp.where(oh, neg, chunks[i]))
        vx1 = fz[:, :1]
        vy1 = fz[:, :1]
        vx2 = fz[:, :1]
        vy2 = fz[:, :1]
        cm_new = cm
        for i in range(b):
            bsel = row1 == i
            ohf = jnp.where(ohs[i], 1.0, 0.0)[None]
            oh4 = jnp.concatenate([ohf] * 4, axis=1)
            bsum = jnp.sum(oh4 * boxc[i],
                           axis=2, keepdims=True)
            bx1 = bsum[:, 0, :]
            by1 = bsum[:, _CS, :]
            bx2 = bsum[:, 2 * _CS, :]
            by2 = bsum[:, 3 * _CS, :]
            for t in range(1, _CS):
                bx1 = bx1 + bsum[:, t, :]
                by1 = by1 + bsum[:, _CS + t, :]
                bx2 = bx2 + bsum[:, 2 * _CS + t, :]
                by2 = by2 + bsum[:, 3 * _CS + t, :]
            vx1 = jnp.where(bsel, bx1, vx1)
            vy1 = jnp.where(bsel, by1, vy1)
            vx2 = jnp.where(bsel, bx2, vx2)
            vy2 = jnp.where(bsel, by2, vy2)
            nmxv = jnp.max(news[i], keepdims=True)
            cm_new = jnp.where((row_ch == i) & (lane_ch == cbs[i]),
                               nmxv, cm_new)
        for i in range(b):
            msk_s[i][pl.ds(cbs[i], 1)] = news[i][None]
        koh = slot == k
        cs = jnp.where(koh, m_vec, cs)
        c1 = jnp.where(koh, vx1, c1)
        c2 = jnp.where(koh, vy1, c2)
        c3 = jnp.where(koh, vx2, c3)
        c4 = jnp.where(koh, vy2, c4)
        return cm_new, cs, c1, c2, c3, c4

    _, cs, c1, c2, c3, c4 = lax.fori_loop(
        0, top_k, ext_body,
        (cm0, jnp.full((b, slots), neg), fz, fz, fz, fz))

    carea = (c3 - c1) * (c4 - c2)
    alive0 = jnp.where(cs > _CONF, 1, 0)
    x1T = c1[:, :, None]
    y1T = c2[:, :, None]
    x2T = c3[:, :, None]
    y2T = c4[:, :, None]
    aT = carea[:, :, None]
    x1B = c1[:, None, :]
    y1B = c2[:, None, :]
    x2B = c3[:, None, :]
    y2B = c4[:, None, :]
    aB = carea[:, None, :]
    ww = jnp.maximum(jnp.minimum(x2T, x2B) - jnp.maximum(x1T, x1B), 0.0)
    hh = jnp.maximum(jnp.minimum(y2T, y2B) - jnp.maximum(y1T, y1B), 0.0)
    inter = ww * hh
    iou = inter / ((aB - inter) + aT)
    sup_s[...] = jnp.where(iou <= _NMS_T, 0, 1)

    supp = jnp.zeros((b, slots), jnp.int32)
    kept = jnp.zeros((b, slots), jnp.int32)
    for s in range(slots):
        keep_s = jnp.where(
            (alive0[:, s:s + 1] > 0) & (supp[:, s:s + 1] == 0), 1, 0)
        supp = supp | jnp.where(keep_s > 0, sup_s[:, s], 0)
        kept = jnp.where(slot == s, keep_s, kept)

    pos = kept
    sh = 1
    while sh < slots:
        pos = pos + jnp.concatenate(
            [jnp.zeros((b, sh), jnp.int32), pos[:, :slots - sh]], axis=1)
        sh *= 2
    pos = pos - kept
    iota_r = lax.broadcasted_iota(jnp.int32, (b, slots, slots), 2)
    perm = jnp.where((pos[:, :, None] == iota_r) & (kept[:, :, None] > 0),
                     1.0, 0.0)
    csz = jnp.where(kept > 0, cs, 0.0)
    rs = jnp.sum(perm * csz[:, :, None], axis=1)
    r1 = jnp.sum(perm * c1[:, :, None], axis=1)
    r2 = jnp.sum(perm * c2[:, :, None], axis=1)
    r3 = jnp.sum(perm * c3[:, :, None], axis=1)
    r4 = jnp.sum(perm * c4[:, :, None], axis=1)

    clf = cl_ref[:, 0:1]
    cls = clf.astype(jnp.int32).reshape(b, 1, 1)
    found = (clf >= 0).reshape(b, 1, 1)
    cmask = (lax.broadcasted_iota(jnp.int32, (b, num_classes, 1), 1) == cls
             ) & found
    os_ref[...] = jnp.where(cmask, rs.reshape(b, 1, slots), 0.0)
    ox1_ref[...] = jnp.where(cmask, r1.reshape(b, 1, slots), 0.0)
    oy1_ref[...] = jnp.where(cmask, r2.reshape(b, 1, slots), 0.0)
    ox2_ref[...] = jnp.where(cmask, r3.reshape(b, 1, slots), 0.0)
    oy2_ref[...] = jnp.where(cmask, r4.reshape(b, 1, slots), 0.0)


@jax.jit
def kernel(loc_data, conf_data, prior_data):
    b, n, _ = loc_data.shape
    num_classes = conf_data.shape[2]
    npad = -(-n // _CH) * _CH
    rows = npad // _LANES
    nch = npad // _CH
    su = -(-_TOP_K // _LANES)
    slots = su * _LANES

    loc_t = jnp.transpose(loc_data, (0, 2, 1))
    pri_t = jnp.transpose(prior_data, (1, 0))
    pad = npad - n
    loc_t = jnp.pad(loc_t, ((0, 0), (0, 0), (0, pad)))
    pri_t = jnp.pad(pri_t, ((0, 0), (0, pad)))
    loc_t = loc_t.reshape(b, 4, rows, _LANES)
    pri_t = pri_t.reshape(4, rows, _LANES)
    conf_p = jnp.pad(conf_data, ((0, 0), (0, pad), (0, 0)))

    nblk = 2 if rows % 2 == 0 else 1
    rowblk = npad // nblk
    cmaxk = functools.partial(_cmax_body, nblk=nblk,
                              num_classes=num_classes)
    cl_sh = jax.ShapeDtypeStruct((b, 1, _LANES), jnp.float32)
    clo = pl.pallas_call(
        cmaxk,
        grid=(b, nblk),
        in_specs=[pl.BlockSpec((1, rowblk, num_classes),
                               lambda i, j: (i, j, 0))],
        out_specs=pl.BlockSpec((1, 1, _LANES), lambda i, j: (i, 0, 0)),
        out_shape=cl_sh,
        scratch_shapes=[pltpu.VMEM((1, num_classes), jnp.float32)],
    )(conf_p)
    clv = clo.reshape(b, _LANES)
    cls_idx = jnp.where(clv[:, 0] >= 0, clv[:, 0], 1.0).astype(jnp.int32)

    onehot = (cls_idx[:, None] == jnp.arange(num_classes)[None]
              ).astype(jnp.float32)
    scores_flat = jnp.sum(conf_p * onehot[:, None, :], axis=2)
    scores_pl = scores_flat.reshape(b, 1, rows, _LANES)

    prep = functools.partial(_prep_body, nch=nch, num_classes=num_classes)
    plane_sh = jax.ShapeDtypeStruct((b, nch, _CS, _LANES), jnp.float32)
    planes = pl.pallas_call(
        prep,
        grid=(b,),
        in_specs=[
            pl.BlockSpec((1, 1, rows, _LANES), lambda i: (i, 0, 0, 0)),
            pl.BlockSpec((1, 4, rows, _LANES), lambda i: (i, 0, 0, 0)),
            pl.BlockSpec((4, rows, _LANES), lambda i: (0, 0, 0)),
        ],
        out_specs=[pl.BlockSpec((1, nch, _CS, _LANES),
                                lambda i: (i, 0, 0, 0))] * 5,
        out_shape=[plane_sh] * 5,
    )(scores_pl, loc_t, pri_t)

    msk, x1p, y1p, x2p, y2p = planes[:5]

    det = functools.partial(_detect_body, b=b, nch=nch,
                            num_classes=num_classes, top_k=_TOP_K,
                            slots=slots)
    out_sh = jax.ShapeDtypeStruct((b, num_classes, slots), jnp.float32)
    outs = pl.pallas_call(
        det,
        out_shape=[out_sh] * 5,
        scratch_shapes=[pltpu.VMEM((b, slots, slots), jnp.int32)]
        + [pltpu.VMEM((nch, _CS, _LANES), jnp.float32)] * b,
    )(msk, x1p, y1p, x2p, y2p, clv)

    stacked = jnp.stack(outs, axis=-1)
    return stacked[:, :, :_TOP_K, :]


# speedup vs baseline: 1.9824x; 1.8987x over previous
import functools
import jax
import jax.numpy as jnp
from jax import lax
from jax.experimental import pallas as pl
from jax.experimental.pallas import tpu as pltpu

_TOP_K = 200
_CONF = 0.1
_NMS_T = 0.45
_V0 = 0.1
_V1 = 0.2
_LANES = 128
_CS = 2
_CH = _CS * _LANES


def _prep_body(loc_ref, conf_ref, pri_ref,
               om_ref, ox1_ref, oy1_ref, ox2_ref, oy2_ref, ocl_ref,
               *, nch, num_classes):
    neg = jnp.float32(-jnp.inf)

    cmax = jnp.max(conf_ref[0], axis=2)
    cmax = jnp.max(cmax, axis=1, keepdims=True)
    iota_c = lax.broadcasted_iota(jnp.int32, (num_classes, 1), 0)
    has = (cmax > _CONF) & (iota_c >= 1)
    cl = jnp.min(jnp.where(has, iota_c, num_classes))
    any_found = cl < num_classes
    cl = jnp.where(any_found, cl, 1)
    clf = jnp.where(any_found, cl, -1)
    ocl_ref[0] = jnp.full((1, _LANES), clf, jnp.float32)

    scores = conf_ref[0, pl.ds(cl, 1)][0]

    lx = loc_ref[0, 0]
    ly = loc_ref[0, 1]
    lw = loc_ref[0, 2]
    lh = loc_ref[0, 3]
    pcx = pri_ref[0]
    pcy = pri_ref[1]
    pw = pri_ref[2]
    ph = pri_ref[3]
    bcx = pcx + lx * _V0 * pw
    bcy = pcy + ly * _V0 * ph
    bw = pw * jnp.exp(lw * _V1)
    bh = ph * jnp.exp(lh * _V1)
    x1 = bcx - bw / 2
    y1 = bcy - bh / 2
    om_ref[0] = jnp.where(scores > _CONF, scores, neg).reshape(
        nch, _CS, _LANES)
    ox1_ref[0] = x1.reshape(nch, _CS, _LANES)
    oy1_ref[0] = y1.reshape(nch, _CS, _LANES)
    ox2_ref[0] = (bw + x1).reshape(nch, _CS, _LANES)
    oy2_ref[0] = (bh + y1).reshape(nch, _CS, _LANES)


def _detect_body(msk_ref, x1_ref, y1_ref, x2_ref, y2_ref, cl_ref,
                 os_ref, ox1_ref, oy1_ref, ox2_ref, oy2_ref,
                 sup_s, *msk_s,
                 b, nch, num_classes, top_k, slots):
    neg = jnp.float32(-jnp.inf)

    for i in range(b):
        msk_s[i][...] = msk_ref[i]
    cm0 = jnp.max(jnp.max(msk_ref[...], axis=3), axis=2)
    lane1 = lax.broadcasted_iota(jnp.int32, (1, nch), 1)
    row_ch = lax.broadcasted_iota(jnp.int32, (b, nch), 0)
    lane_ch = lax.broadcasted_iota(jnp.int32, (b, nch), 1)
    lin = (lax.broadcasted_iota(jnp.int32, (_CS, _LANES), 0) * _LANES
           + lax.broadcasted_iota(jnp.int32, (_CS, _LANES), 1))
    slot = lax.broadcasted_iota(jnp.int32, (b, slots), 1)
    fz = jnp.zeros((b, slots), jnp.float32)
    row1 = lax.broadcasted_iota(jnp.int32, (b, 1), 0)

    def ext_body(k, carry):
        cm, m_vec, cs, c1, c2, c3, c4, *cbs = carry
        chunks = [msk_s[i][pl.ds(cbs[i], 1)][0] for i in range(b)]
        boxc = [jnp.concatenate(
            [x1_ref[i, pl.ds(cbs[i], 1)],
             y1_ref[i, pl.ds(cbs[i], 1)],
             x2_ref[i, pl.ds(cbs[i], 1)],
             y2_ref[i, pl.ds(cbs[i], 1)]], axis=1) for i in range(b)]
        ohs = []
        news = []
        for i in range(b):
            liv = jnp.max(jnp.where(chunks[i] == m_vec[i:i + 1], lin, -1),
                          keepdims=True)
            oh = lin == liv
            ohs.append(oh)
            news.append(jnp.where(oh, neg, chunks[i]))
        vx1 = fz[:, :1]
        vy1 = fz[:, :1]
        vx2 = fz[:, :1]
        vy2 = fz[:, :1]
        cm_new = cm
        for i in range(b):
            bsel = row1 == i
            ohf = jnp.where(ohs[i], 1.0, 0.0)[None]
            oh4 = jnp.concatenate([ohf] * 4, axis=1)
            bsum = jnp.sum(oh4 * boxc[i],
                           axis=2, keepdims=True)
            bx1 = bsum[:, 0, :]
            by1 = bsum[:, _CS, :]
            bx2 = bsum[:, 2 * _CS, :]
            by2 = bsum[:, 3 * _CS, :]
            for t in range(1, _CS):
                bx1 = bx1 + bsum[:, t, :]
                by1 = by1 + bsum[:, _CS + t, :]
                bx2 = bx2 + bsum[:, 2 * _CS + t, :]
                by2 = by2 + bsum[:, 3 * _CS + t, :]
            vx1 = jnp.where(bsel, bx1, vx1)
            vy1 = jnp.where(bsel, by1, vy1)
            vx2 = jnp.where(bsel, bx2, vx2)
            vy2 = jnp.where(bsel, by2, vy2)
            nmxv = jnp.max(news[i], keepdims=True)
            cm_new = jnp.where((row_ch == i) & (lane_ch == cbs[i]),
                               nmxv, cm_new)
        for i in range(b):
            msk_s[i][pl.ds(cbs[i], 1)] = news[i][None]
        koh = slot == k
        cs = jnp.where(koh, m_vec, cs)
        c1 = jnp.where(koh, vx1, c1)
        c2 = jnp.where(koh, vy1, c2)
        c3 = jnp.where(koh, vx2, c3)
        c4 = jnp.where(koh, vy2, c4)
        m_next = jnp.max(cm_new, axis=1, keepdims=True)
        cbs_next = [jnp.max(jnp.where(
            cm_new[i:i + 1] == m_next[i:i + 1], lane1, -1))
            for i in range(b)]
        return (cm_new, m_next, cs, c1, c2, c3, c4, *cbs_next)

    m_vec0 = jnp.max(cm0, axis=1, keepdims=True)
    cbs0 = [jnp.max(jnp.where(cm0[i:i + 1] == m_vec0[i:i + 1], lane1, -1))
            for i in range(b)]
    res = lax.fori_loop(
        0, top_k, ext_body,
        (cm0, m_vec0, jnp.full((b, slots), neg), fz, fz, fz, fz, *cbs0))
    cs, c1, c2, c3, c4 = res[2:7]

    carea = (c3 - c1) * (c4 - c2)
    alive0 = jnp.where(cs > _CONF, 1, 0)
    x1T = c1[:, :, None]
    y1T = c2[:, :, None]
    x2T = c3[:, :, None]
    y2T = c4[:, :, None]
    aT = carea[:, :, None]
    x1B = c1[:, None, :]
    y1B = c2[:, None, :]
    x2B = c3[:, None, :]
    y2B = c4[:, None, :]
    aB = carea[:, None, :]
    ww = jnp.maximum(jnp.minimum(x2T, x2B) - jnp.maximum(x1T, x1B), 0.0)
    hh = jnp.maximum(jnp.minimum(y2T, y2B) - jnp.maximum(y1T, y1B), 0.0)
    inter = ww * hh
    iou = inter / ((aB - inter) + aT)
    sup_s[...] = jnp.where(iou <= _NMS_T, 0, 1)

    supp = jnp.zeros((b, slots), jnp.int32)
    kept = jnp.zeros((b, slots), jnp.int32)
    for s in range(slots):
        keep_s = jnp.where(
            (alive0[:, s:s + 1] > 0) & (supp[:, s:s + 1] == 0), 1, 0)
        supp = supp | jnp.where(keep_s > 0, sup_s[:, s], 0)
        kept = jnp.where(slot == s, keep_s, kept)

    pos = kept
    sh = 1
    while sh < slots:
        pos = pos + jnp.concatenate(
            [jnp.zeros((b, sh), jnp.int32), pos[:, :slots - sh]], axis=1)
        sh *= 2
    pos = pos - kept
    iota_r = lax.broadcasted_iota(jnp.int32, (b, slots, slots), 2)
    perm = jnp.where((pos[:, :, None] == iota_r) & (kept[:, :, None] > 0),
                     1.0, 0.0)
    csz = jnp.where(kept > 0, cs, 0.0)
    rs = jnp.sum(perm * csz[:, :, None], axis=1)
    r1 = jnp.sum(perm * c1[:, :, None], axis=1)
    r2 = jnp.sum(perm * c2[:, :, None], axis=1)
    r3 = jnp.sum(perm * c3[:, :, None], axis=1)
    r4 = jnp.sum(perm * c4[:, :, None], axis=1)

    clf = cl_ref[:, 0:1]
    cls = clf.astype(jnp.int32).reshape(b, 1, 1)
    found = (clf >= 0).reshape(b, 1, 1)
    cmask = (lax.broadcasted_iota(jnp.int32, (b, num_classes, 1), 1) == cls
             ) & found
    os_ref[...] = jnp.where(cmask, rs.reshape(b, 1, slots), 0.0)
    ox1_ref[...] = jnp.where(cmask, r1.reshape(b, 1, slots), 0.0)
    oy1_ref[...] = jnp.where(cmask, r2.reshape(b, 1, slots), 0.0)
    ox2_ref[...] = jnp.where(cmask, r3.reshape(b, 1, slots), 0.0)
    oy2_ref[...] = jnp.where(cmask, r4.reshape(b, 1, slots), 0.0)


@jax.jit
def kernel(loc_data, conf_data, prior_data):
    b, n, _ = loc_data.shape
    num_classes = conf_data.shape[2]
    npad = -(-n // _CH) * _CH
    rows = npad // _LANES
    nch = npad // _CH
    su = -(-_TOP_K // _LANES)
    slots = su * _LANES

    loc_t = jnp.transpose(loc_data, (0, 2, 1))
    conf_t = jnp.transpose(conf_data, (0, 2, 1))
    pri_t = jnp.transpose(prior_data, (1, 0))
    pad = npad - n
    loc_t = jnp.pad(loc_t, ((0, 0), (0, 0), (0, pad)))
    conf_t = jnp.pad(conf_t, ((0, 0), (0, 0), (0, pad)))
    pri_t = jnp.pad(pri_t, ((0, 0), (0, pad)))
    loc_t = loc_t.reshape(b, 4, rows, _LANES)
    conf_t = conf_t.reshape(b, num_classes, rows, _LANES)
    pri_t = pri_t.reshape(4, rows, _LANES)

    prep = functools.partial(_prep_body, nch=nch, num_classes=num_classes)
    plane_sh = jax.ShapeDtypeStruct((b, nch, _CS, _LANES), jnp.float32)
    cl_sh = jax.ShapeDtypeStruct((b, 1, _LANES), jnp.float32)
    planes = pl.pallas_call(
        prep,
        grid=(b,),
        in_specs=[
            pl.BlockSpec((1, 4, rows, _LANES), lambda i: (i, 0, 0, 0)),
            pl.BlockSpec((1, num_classes, rows, _LANES),
                         lambda i: (i, 0, 0, 0)),
            pl.BlockSpec((4, rows, _LANES), lambda i: (0, 0, 0)),
        ],
        out_specs=[pl.BlockSpec((1, nch, _CS, _LANES),
                                lambda i: (i, 0, 0, 0))] * 5
        + [pl.BlockSpec((1, 1, _LANES), lambda i: (i, 0, 0))],
        out_shape=[plane_sh] * 5 + [cl_sh],
    )(loc_t, conf_t, pri_t)

    msk, x1p, y1p, x2p, y2p = planes[:5]
    clv = planes[5].reshape(b, _LANES)

    det = functools.partial(_detect_body, b=b, nch=nch,
                            num_classes=num_classes, top_k=_TOP_K,
                            slots=slots)
    out_sh = jax.ShapeDtypeStruct((b, num_classes, slots), jnp.float32)
    outs = pl.pallas_call(
        det,
        out_shape=[out_sh] * 5,
        scratch_shapes=[pltpu.VMEM((b, slots, slots), jnp.int32)]
        + [pltpu.VMEM((nch, _CS, _LANES), jnp.float32)] * b,
    )(msk, x1p, y1p, x2p, y2p, clv)

    stacked = jnp.stack(outs, axis=-1)
    return stacked[:, :, :_TOP_K, :]
